# baseline (speedup 1.0000x reference)
import functools

import jax
import jax.numpy as jnp
from jax import lax
from jax.experimental import pallas as pl
from jax.experimental.pallas import tpu as pltpu
from jax.experimental.pallas import tpu_sc as plsc

E = 16
K = 4
S = 2048
D = 2048
DFF = 768

NC = 2
NS = 16
NW = NC * NS
TPW = S // NW
BLK = 256
P = 12288
MAXB = P // BLK
DCH = 256
NDC = D // DCH


def _router_body(x_ref, gw_ref, eid_ref, wv_ref):
    x = x_ref[...]
    logits = jax.lax.dot_general(
        x, gw_ref[...], (((1,), (1,)), ((), ())),
        preferred_element_type=jnp.float32)
    m = jnp.max(logits, axis=-1, keepdims=True)
    ex = jnp.exp(logits - m)
    probs = ex / jnp.sum(ex, axis=-1, keepdims=True)
    p = probs
    idx = jax.lax.broadcasted_iota(jnp.int32, probs.shape, 1)
    ids = []
    vals = []
    for _ in range(K):
        cur = jnp.max(p, axis=-1, keepdims=True)
        amax = jnp.min(jnp.where(p == cur, idx, E), axis=-1, keepdims=True)
        mask = idx == amax
        p = jnp.where(mask, -jnp.inf, p)
        ids.append(amax)
        vals.append(cur)
    ids = jnp.concatenate(ids, axis=1)
    vals = jnp.concatenate(vals, axis=1)
    denom = jnp.clip(jnp.sum(vals, axis=-1, keepdims=True), 1e-12, None)
    eid_ref[...] = ids
    wv_ref[...] = vals / denom


def _dispatch_body(eids_hbm, wv_hbm, xb_hbm,
                   xs_out, ws_out, inv_out, be_out, nb_out,
                   eids_v, wv_v, hist_all, hist_pre, wbase_v, ps_v,
                   pos2_v, posflat_v, wk_v, rows_v, blk_v, nb_v):
    cid = lax.axis_index("c")
    sid = lax.axis_index("s")
    wid = sid * NC + cid
    t0 = wid * TPW
    iota = lax.iota(jnp.int32, 16)
    m4 = iota < 4
    lane4 = iota & 3
    ones = jnp.ones((16,), jnp.int32)

    pltpu.sync_copy(eids_hbm, eids_v)
    pltpu.sync_copy(wv_hbm, wv_v)

    hist_all[...] = jnp.zeros((16,), jnp.int32)
    hist_pre[...] = jnp.zeros((16,), jnp.int32)

    def hist_step(i, carry):
        idvec = plsc.load_gather(eids_v, [i * 4 + lane4], mask=m4) & 15
        plsc.addupdate_scatter(hist_all, [idvec], ones, mask=m4)
        plsc.addupdate_scatter(hist_pre, [idvec], ones, mask=m4 & (i < t0))
        return carry
    lax.fori_loop(0, S, hist_step, 0)

    total = hist_all[...]
    padded = ((total + (BLK - 1)) >> 8) << 8
    incl = plsc.cumsum(padded)
    ps = incl - padded
    ps_v[...] = ps
    wbase_v[...] = ps + hist_pre[...]

    def pos_step(ii, carry):
        i = t0 + ii
        idvec = plsc.load_gather(eids_v, [i * 4 + lane4], mask=m4) & 15
        pos4 = plsc.load_gather(wbase_v, [idvec], mask=m4)
        plsc.store_scatter(pos2_v, [lane4, iota * 0 + ii], pos4, mask=m4)
        plsc.addupdate_scatter(wbase_v, [idvec], ones, mask=m4)
        return carry
    lax.fori_loop(0, TPW, pos_step, 0)

    for k2 in range(K):
        for jj in range(TPW // 16):
            posflat_v[pl.ds(k2 * TPW + jj * 16, 16)] = (
                pos2_v[k2, pl.ds(jj * 16, 16)])
    pltpu.sync_copy(posflat_v, inv_out.at[pl.ds(wid * K * TPW, K * TPW)])

    @pl.when(wid == 0)
    def _():
        for jj in range(MAXB // 16):
            bvals = (jj * 16 + iota) * BLK
            cnt = jnp.zeros((16,), jnp.int32)
            for e in range(E):
                se = jnp.sum(jnp.where(iota == e, ps, 0))
                cnt += (bvals >= se).astype(jnp.int32)
            blk_v[pl.ds(jj * 16, 16)] = cnt - 1
        pltpu.sync_copy(blk_v, be_out)
        nb_scalar = jnp.max(incl) // BLK
        nb_v[...] = iota * 0 + nb_scalar
        pltpu.sync_copy(nb_v, nb_out)

    pltpu.sync_copy(xb_hbm.at[pl.ds(t0, TPW)], rows_v)
    for k in range(K):
        pltpu.sync_copy(rows_v, xs_out.at[pos2_v.at[k]])
        for jj in range(TPW // 16):
            widx = (t0 + jj * 16 + iota) * 4 + k
            wk_v[pl.ds(jj * 16, 16)] = plsc.load_gather(wv_v, [widx])
        pltpu.sync_copy(wk_v, ws_out.at[pos2_v.at[k]])


def _gemm_body(be_ref, nb_ref, ws_ref, xs_ref, gup_ref, down_ref, y_ref):
    b = pl.program_id(0)

    @pl.when(b < nb_ref[0])
    def _():
        x = xs_ref[...]
        gu = jax.lax.dot_general(
            x, gup_ref[0], (((1,), (1,)), ((), ())),
            preferred_element_type=jnp.float32)
        gate = gu[:, :DFF]
        up = gu[:, DFF:]
        h = (gate * jax.lax.logistic(gate) * up).astype(jnp.bfloat16)
        dout = jax.lax.dot_general(
            h, down_ref[0], (((1,), (1,)), ((), ())),
            preferred_element_type=jnp.float32)
        y_ref[...] = ws_ref[...] * dout


def _combine_body(y4_hbm, inv_hbm, out_hbm,
                  inv_v, idx_v, yrows_v, outbuf_v, sem):
    cid = lax.axis_index("c")
    sid = lax.axis_index("s")
    wid = sid * NC + cid
    t0 = wid * TPW
    iota = lax.iota(jnp.int32, 16)

    pltpu.sync_copy(inv_hbm.at[pl.ds(wid * K * TPW, K * TPW)], inv_v)
    for c in range(NDC):
        for jj in range((K * TPW) // 16):
            iv = inv_v[pl.ds(jj * 16, 16)]
            idx_v[pl.ds(jj * 16, 16)] = iv * NDC + c
        pltpu.async_copy(y4_hbm.at[idx_v], yrows_v, sem).wait()

        def acc_step(m, carry):
            ii = m >> 4
            q = m & 15
            a = (yrows_v[0 * TPW + ii, pl.ds(q * 16, 16)]
                 + yrows_v[1 * TPW + ii, pl.ds(q * 16, 16)]
                 + yrows_v[2 * TPW + ii, pl.ds(q * 16, 16)]
                 + yrows_v[3 * TPW + ii, pl.ds(q * 16, 16)])
            outbuf_v[ii, pl.ds(q * 16, 16)] = a
            return carry
        lax.fori_loop(0, TPW * (DCH // 16), acc_step, 0)
        pltpu.sync_copy(outbuf_v,
                        out_hbm.at[pl.ds(t0, TPW), pl.ds(c * DCH, DCH)])


def _sc_mesh():
    return plsc.VectorSubcoreMesh(core_axis_name="c", subcore_axis_name="s",
                                  num_cores=NC, num_subcores=NS)


def _make_dispatch():
    mesh = _sc_mesh()
    return functools.partial(
        pl.kernel,
        out_type=(jax.ShapeDtypeStruct((P, 8, 128), jnp.int32),
                  jax.ShapeDtypeStruct((P,), jnp.float32),
                  jax.ShapeDtypeStruct((NW * K * TPW,), jnp.int32),
                  jax.ShapeDtypeStruct((MAXB,), jnp.int32),
                  jax.ShapeDtypeStruct((16,), jnp.int32)),
        mesh=mesh,
        compiler_params=pltpu.CompilerParams(needs_layout_passes=False),
        scratch_types=[
            pltpu.VMEM((S * K,), jnp.int32),
            pltpu.VMEM((S * K,), jnp.float32),
            pltpu.VMEM((16,), jnp.int32),
            pltpu.VMEM((16,), jnp.int32),
            pltpu.VMEM((16,), jnp.int32),
            pltpu.VMEM((16,), jnp.int32),
            pltpu.VMEM((K, TPW), jnp.int32),
            pltpu.VMEM((K * TPW,), jnp.int32),
            pltpu.VMEM((TPW,), jnp.float32),
            pltpu.VMEM((TPW, 8, 128), jnp.int32),
            pltpu.VMEM((MAXB,), jnp.int32),
            pltpu.VMEM((16,), jnp.int32),
        ],
    )(_dispatch_body)


def _make_combine():
    mesh = _sc_mesh()
    return functools.partial(
        pl.kernel,
        out_type=jax.ShapeDtypeStruct((S, D), jnp.float32),
        mesh=mesh,
        compiler_params=pltpu.CompilerParams(needs_layout_passes=False),
        scratch_types=[
            pltpu.VMEM((K * TPW,), jnp.int32),
            pltpu.VMEM((K * TPW,), jnp.int32),
            pltpu.VMEM((K * TPW, DCH), jnp.float32),
            pltpu.VMEM((TPW, DCH), jnp.float32),
            pltpu.SemaphoreType.DMA,
        ],
    )(_combine_body)


def kernel(hidden_states, gate_weight, gate_up_proj, down_proj):
    b, s, d = hidden_states.shape
    x = hidden_states.reshape(s, d)

    eids, wv = pl.pallas_call(
        _router_body,
        out_shape=(jax.ShapeDtypeStruct((S, K), jnp.int32),
                   jax.ShapeDtypeStruct((S, K), jnp.float32)),
    )(x, gate_weight)

    xbb = x.astype(jnp.bfloat16)
    xb = jax.lax.bitcast_convert_type(
        xbb.reshape(S, 1024, 2), jnp.int32).reshape(S, 8, 128)
    eids_f = eids.reshape(S * K)
    wv_f = wv.reshape(S * K)

    xs, ws, inv, be, nb = _make_dispatch()(eids_f, wv_f, xb)

    gupb = gate_up_proj.astype(jnp.bfloat16)
    downb = down_proj.astype(jnp.bfloat16)
    xs2 = jax.lax.bitcast_convert_type(
        xs.reshape(P, D // 2), jnp.bfloat16).reshape(P, D)
    ws2 = ws.reshape(P, 1)

    grid_spec = pltpu.PrefetchScalarGridSpec(
        num_scalar_prefetch=2,
        grid=(MAXB,),
        in_specs=[
            pl.BlockSpec((BLK, 1),
                         lambda i, be, nb: (jnp.minimum(i, nb[0] - 1), 0)),
            pl.BlockSpec((BLK, D),
                         lambda i, be, nb: (jnp.minimum(i, nb[0] - 1), 0)),
            pl.BlockSpec((1, 2 * DFF, D), lambda i, be, nb: (be[i], 0, 0)),
            pl.BlockSpec((1, D, DFF), lambda i, be, nb: (be[i], 0, 0)),
        ],
        out_specs=pl.BlockSpec(
            (BLK, D), lambda i, be, nb: (jnp.minimum(i, nb[0] - 1), 0)),
    )
    y = pl.pallas_call(
        _gemm_body,
        grid_spec=grid_spec,
        out_shape=jax.ShapeDtypeStruct((P, D), jnp.float32),
    )(be, nb, ws2, xs2, gupb, downb)

    y4 = y.reshape(P * NDC, DCH)
    out = _make_combine()(y4, inv)
    return out.reshape(b, s, d)


# speedup vs baseline: 2.4635x; 2.4635x over previous
import functools

import jax
import jax.numpy as jnp
from jax import lax
from jax.experimental import pallas as pl
from jax.experimental.pallas import tpu as pltpu
from jax.experimental.pallas import tpu_sc as plsc

E = 16
K = 4
S = 2048
D = 2048
DFF = 768

NC = 2
NS = 16
NW = NC * NS
TPW = S // NW
BLK = 256
P = 12288
MAXB = P // BLK
DCH = 256
NDC = D // DCH


def _router_body(x_ref, gw_ref, eid_ref, wv_ref):
    x = x_ref[...]
    logits = jax.lax.dot_general(
        x, gw_ref[...], (((1,), (1,)), ((), ())),
        preferred_element_type=jnp.float32)
    m = jnp.max(logits, axis=-1, keepdims=True)
    ex = jnp.exp(logits - m)
    probs = ex / jnp.sum(ex, axis=-1, keepdims=True)
    p = probs
    idx = jax.lax.broadcasted_iota(jnp.int32, probs.shape, 1)
    ids = []
    vals = []
    for _ in range(K):
        cur = jnp.max(p, axis=-1, keepdims=True)
        amax = jnp.min(jnp.where(p == cur, idx, E), axis=-1, keepdims=True)
        mask = idx == amax
        p = jnp.where(mask, -jnp.inf, p)
        ids.append(amax)
        vals.append(cur)
    ids = jnp.concatenate(ids, axis=1)
    vals = jnp.concatenate(vals, axis=1)
    denom = jnp.clip(jnp.sum(vals, axis=-1, keepdims=True), 1e-12, None)
    eid_ref[...] = ids
    wv_ref[...] = vals / denom


def _dispatch_body(eids_hbm, wv_hbm, xb_hbm,
                   xs_out, ws_out, inv_out, be_out, nb_out,
                   eids_v, wv_v, hist_all, hist_pre, wbase_v, ps_v,
                   pos2_v, posflat_v, wk_v, rows_v, blk_v, nb_v):
    cid = lax.axis_index("c")
    sid = lax.axis_index("s")
    wid = sid * NC + cid
    t0 = wid * TPW
    iota = lax.iota(jnp.int32, 16)
    m4 = iota < 4
    lane4 = iota & 3
    ones = jnp.ones((16,), jnp.int32)

    pltpu.sync_copy(eids_hbm, eids_v)
    pltpu.sync_copy(wv_hbm, wv_v)

    hist_all[...] = jnp.zeros((16,), jnp.int32)
    hist_pre[...] = jnp.zeros((16,), jnp.int32)

    def hist_step(i, carry):
        idvec = plsc.load_gather(eids_v, [i * 4 + lane4], mask=m4) & 15
        plsc.addupdate_scatter(hist_all, [idvec], ones, mask=m4)
        plsc.addupdate_scatter(hist_pre, [idvec], ones, mask=m4 & (i < t0))
        return carry
    lax.fori_loop(0, S, hist_step, 0)

    total = hist_all[...]
    padded = ((total + (BLK - 1)) >> 8) << 8
    incl = plsc.cumsum(padded)
    ps = incl - padded
    ps_v[...] = ps
    wbase_v[...] = ps + hist_pre[...]

    def pos_step(ii, carry):
        i = t0 + ii
        idvec = plsc.load_gather(eids_v, [i * 4 + lane4], mask=m4) & 15
        pos4 = plsc.load_gather(wbase_v, [idvec], mask=m4)
        plsc.store_scatter(pos2_v, [lane4 * 2 + ii // 32, iota * 0 + (ii & 31)],
                           pos4, mask=m4)
        plsc.store_scatter(posflat_v, [ii * 4 + lane4], pos4, mask=m4)
        plsc.addupdate_scatter(wbase_v, [idvec], ones, mask=m4)
        return carry
    lax.fori_loop(0, TPW, pos_step, 0)

    pltpu.sync_copy(posflat_v, inv_out.at[pl.ds(wid * K * TPW, K * TPW)])

    @pl.when(wid == 0)
    def _():
        for jj in range(MAXB // 16):
            bvals = (jj * 16 + iota) * BLK
            cnt = jnp.zeros((16,), jnp.int32)
            for e in range(E):
                se = jnp.sum(jnp.where(iota == e, ps, 0))
                cnt += (bvals >= se).astype(jnp.int32)
            blk_v[pl.ds(jj * 16, 16)] = cnt - 1
        pltpu.sync_copy(blk_v, be_out)
        nb_scalar = jnp.max(incl) // BLK
        nb_v[...] = iota * 0 + nb_scalar
        pltpu.sync_copy(nb_v, nb_out)

    for h in range(2):
        pltpu.sync_copy(xb_hbm.at[pl.ds(t0 + h * 32, 32)], rows_v)
        for k in range(K):
            pltpu.sync_copy(rows_v, xs_out.at[pos2_v.at[k * 2 + h]])
    for k in range(K):
        for h in range(2):
            for jj in range(2):
                widx = (t0 + h * 32 + jj * 16 + iota) * 4 + k
                wk_v[pl.ds(jj * 16, 16)] = plsc.load_gather(wv_v, [widx])
            pltpu.sync_copy(wk_v, ws_out.at[pos2_v.at[k * 2 + h]])


def _gemm_body(be_ref, nb_ref, ws_ref, xs_ref, gup_ref, down_ref, y_ref):
    b = pl.program_id(0)

    @pl.when(b < nb_ref[0])
    def _():
        x = xs_ref[...].astype(jnp.bfloat16)
        gu = jax.lax.dot_general(
            x, gup_ref[0], (((1,), (1,)), ((), ())),
            preferred_element_type=jnp.float32)
        gate = gu[:, :DFF]
        up = gu[:, DFF:]
        h = (gate * jax.lax.logistic(gate) * up).astype(jnp.bfloat16)
        dout = jax.lax.dot_general(
            h, down_ref[0], (((1,), (1,)), ((), ())),
            preferred_element_type=jnp.float32)
        y_ref[...] = ws_ref[...] * dout


TG = 8


def _combine_body(y_hbm, inv_hbm, out_hbm,
                  inv_v, yrows_v, outbuf_v, sem):
    cid = lax.axis_index("c")
    sid = lax.axis_index("s")
    wid = sid * NC + cid
    t0 = wid * TPW

    pltpu.sync_copy(inv_hbm.at[pl.ds(wid * K * TPW, K * TPW)], inv_v)
    for g in range(TPW // TG):
        pltpu.async_copy(
            y_hbm.at[inv_v.at[pl.ds(g * TG * K, TG * K)]], yrows_v, sem
        ).wait()

        def acc_step(m, carry):
            ii = m >> 7
            q = m & 127
            a = (yrows_v[ii * 4 + 0, pl.ds(q * 16, 16)]
                 + yrows_v[ii * 4 + 1, pl.ds(q * 16, 16)]
                 + yrows_v[ii * 4 + 2, pl.ds(q * 16, 16)]
                 + yrows_v[ii * 4 + 3, pl.ds(q * 16, 16)])
            outbuf_v[ii, pl.ds(q * 16, 16)] = a
            return carry
        lax.fori_loop(0, TG * (D // 16), acc_step, 0)
        pltpu.sync_copy(outbuf_v, out_hbm.at[pl.ds(t0 + g * TG, TG), :])


def _sc_mesh():
    return plsc.VectorSubcoreMesh(core_axis_name="c", subcore_axis_name="s",
                                  num_cores=NC, num_subcores=NS)


def _make_dispatch():
    mesh = _sc_mesh()
    return functools.partial(
        pl.kernel,
        out_type=(jax.ShapeDtypeStruct((P, D), jnp.float32),
                  jax.ShapeDtypeStruct((P,), jnp.float32),
                  jax.ShapeDtypeStruct((NW * K * TPW,), jnp.int32),
                  jax.ShapeDtypeStruct((MAXB,), jnp.int32),
                  jax.ShapeDtypeStruct((16,), jnp.int32)),
        mesh=mesh,
        compiler_params=pltpu.CompilerParams(needs_layout_passes=False),
        scratch_types=[
            pltpu.VMEM((S * K,), jnp.int32),
            pltpu.VMEM((S * K,), jnp.float32),
            pltpu.VMEM((16,), jnp.int32),
            pltpu.VMEM((16,), jnp.int32),
            pltpu.VMEM((16,), jnp.int32),
            pltpu.VMEM((16,), jnp.int32),
            pltpu.VMEM((K * 2, TPW // 2), jnp.int32),
            pltpu.VMEM((K * TPW,), jnp.int32),
            pltpu.VMEM((TPW // 2,), jnp.float32),
            pltpu.VMEM((TPW // 2, D), jnp.float32),
            pltpu.VMEM((MAXB,), jnp.int32),
            pltpu.VMEM((16,), jnp.int32),
        ],
    )(_dispatch_body)


def _make_combine():
    mesh = _sc_mesh()
    return functools.partial(
        pl.kernel,
        out_type=jax.ShapeDtypeStruct((S, D), jnp.float32),
        mesh=mesh,
        compiler_params=pltpu.CompilerParams(needs_layout_passes=False),
        scratch_types=[
            pltpu.VMEM((K * TPW,), jnp.int32),
            pltpu.VMEM((TG * K, D), jnp.float32),
            pltpu.VMEM((TG, D), jnp.float32),
            pltpu.SemaphoreType.DMA,
        ],
    )(_combine_body)


def kernel(hidden_states, gate_weight, gate_up_proj, down_proj):
    b, s, d = hidden_states.shape
    x = hidden_states.reshape(s, d)

    eids, wv = pl.pallas_call(
        _router_body,
        out_shape=(jax.ShapeDtypeStruct((S, K), jnp.int32),
                   jax.ShapeDtypeStruct((S, K), jnp.float32)),
    )(x, gate_weight)

    eids_f = eids.reshape(S * K)
    wv_f = wv.reshape(S * K)

    xs, ws, inv, be, nb = _make_dispatch()(eids_f, wv_f, x)

    gupb = gate_up_proj.astype(jnp.bfloat16)
    downb = down_proj.astype(jnp.bfloat16)
    ws2 = ws.reshape(P, 1)

    grid_spec = pltpu.PrefetchScalarGridSpec(
        num_scalar_prefetch=2,
        grid=(MAXB,),
        in_specs=[
            pl.BlockSpec((BLK, 1),
                         lambda i, be, nb: (jnp.minimum(i, nb[0] - 1), 0)),
            pl.BlockSpec((BLK, D),
                         lambda i, be, nb: (jnp.minimum(i, nb[0] - 1), 0)),
            pl.BlockSpec((1, 2 * DFF, D), lambda i, be, nb: (be[i], 0, 0)),
            pl.BlockSpec((1, D, DFF), lambda i, be, nb: (be[i], 0, 0)),
        ],
        out_specs=pl.BlockSpec(
            (BLK, D), lambda i, be, nb: (jnp.minimum(i, nb[0] - 1), 0)),
    )
    y = pl.pallas_call(
        _gemm_body,
        grid_spec=grid_spec,
        out_shape=jax.ShapeDtypeStruct((P, D), jnp.float32),
    )(be, nb, ws2, xs, gupb, downb)

    out = _make_combine()(y, inv)
    return out.reshape(b, s, d)


# speedup vs baseline: 2.6058x; 1.0578x over previous
import functools

import jax
import jax.numpy as jnp
from jax import lax
from jax.experimental import pallas as pl
from jax.experimental.pallas import tpu as pltpu
from jax.experimental.pallas import tpu_sc as plsc

E = 16
K = 4
S = 2048
D = 2048
DFF = 768

NC = 2
NS = 16
NW = NC * NS
TPW = S // NW
BLK = 256
P = 12288
MAXB = P // BLK
DCH = 256
NDC = D // DCH


def _router_body(x_ref, gw_ref, eid_ref, wv_ref):
    x = x_ref[...]
    logits = jax.lax.dot_general(
        x, gw_ref[...], (((1,), (1,)), ((), ())),
        preferred_element_type=jnp.float32)
    m = jnp.max(logits, axis=-1, keepdims=True)
    ex = jnp.exp(logits - m)
    probs = ex / jnp.sum(ex, axis=-1, keepdims=True)
    p = probs
    idx = jax.lax.broadcasted_iota(jnp.int32, probs.shape, 1)
    ids = []
    vals = []
    for _ in range(K):
        cur = jnp.max(p, axis=-1, keepdims=True)
        amax = jnp.min(jnp.where(p == cur, idx, E), axis=-1, keepdims=True)
        mask = idx == amax
        p = jnp.where(mask, -jnp.inf, p)
        ids.append(amax)
        vals.append(cur)
    ids = jnp.concatenate(ids, axis=1)
    vals = jnp.concatenate(vals, axis=1)
    denom = jnp.clip(jnp.sum(vals, axis=-1, keepdims=True), 1e-12, None)
    eid_ref[...] = ids
    wv_ref[...] = vals / denom


XCH = 16
NXC = TPW // XCH


def _dispatch_body(eids_hbm, wv_hbm, xb_hbm,
                   xs_out, ws_out, inv_out, be_out, nb_out,
                   eids_v, wv_v, hist_all, hist_pre, wbase_v, ps_v,
                   pos2_v, posflat_v, wk_v, rows_a, rows_b, blk_v, nb_v,
                   lsem, ssem):
    cid = lax.axis_index("c")
    sid = lax.axis_index("s")
    wid = sid * NC + cid
    t0 = wid * TPW
    iota = lax.iota(jnp.int32, 16)
    m4 = iota < 4
    lane4 = iota & 3
    ones = jnp.ones((16,), jnp.int32)
    bufs = [rows_a, rows_b]

    h_load = [None] * NXC
    for c in range(2):
        h_load[c] = pltpu.async_copy(
            xb_hbm.at[pl.ds(t0 + c * XCH, XCH)], bufs[c], lsem)

    pltpu.sync_copy(eids_hbm, eids_v)
    pltpu.sync_copy(wv_hbm, wv_v)

    hist_all[...] = jnp.zeros((16,), jnp.int32)
    hist_pre[...] = jnp.zeros((16,), jnp.int32)

    def hist_step(i, carry):
        idvec = plsc.load_gather(eids_v, [i * 4 + lane4], mask=m4) & 15
        plsc.addupdate_scatter(hist_all, [idvec], ones, mask=m4)
        plsc.addupdate_scatter(hist_pre, [idvec], ones, mask=m4 & (i < t0))
        return carry
    lax.fori_loop(0, S, hist_step, 0)

    total = hist_all[...]
    padded = ((total + (BLK - 1)) >> 8) << 8
    incl = plsc.cumsum(padded)
    ps = incl - padded
    ps_v[...] = ps
    wbase_v[...] = ps + hist_pre[...]

    def pos_step(ii, carry):
        i = t0 + ii
        idvec = plsc.load_gather(eids_v, [i * 4 + lane4], mask=m4) & 15
        pos4 = plsc.load_gather(wbase_v, [idvec], mask=m4)
        plsc.store_scatter(pos2_v,
                           [lane4 * NXC + ii // XCH, iota * 0 + (ii & (XCH - 1))],
                           pos4, mask=m4)
        plsc.store_scatter(posflat_v, [ii * 4 + lane4], pos4, mask=m4)
        plsc.addupdate_scatter(wbase_v, [idvec], ones, mask=m4)
        return carry
    lax.fori_loop(0, TPW, pos_step, 0)

    pltpu.sync_copy(posflat_v, inv_out.at[pl.ds(wid * K * TPW, K * TPW)])

    @pl.when(wid == 0)
    def _():
        for jj in range(MAXB // 16):
            bvals = (jj * 16 + iota) * BLK
            cnt = jnp.zeros((16,), jnp.int32)
            for e in range(E):
                se = jnp.sum(jnp.where(iota == e, ps, 0))
                cnt += (bvals >= se).astype(jnp.int32)
            blk_v[pl.ds(jj * 16, 16)] = cnt - 1
        pltpu.sync_copy(blk_v, be_out)
        nb_scalar = jnp.max(incl) // BLK
        nb_v[...] = iota * 0 + nb_scalar
        pltpu.sync_copy(nb_v, nb_out)

    h_scat = [[None] * K for _ in range(NXC)]
    for c in range(NXC):
        h_load[c].wait()
        for k in range(K):
            h_scat[c][k] = pltpu.async_copy(
                bufs[c % 2], xs_out.at[pos2_v.at[k * NXC + c]], ssem)
        if c + 2 < NXC:
            for k in range(K):
                h_scat[c][k].wait()
            h_load[c + 2] = pltpu.async_copy(
                xb_hbm.at[pl.ds(t0 + (c + 2) * XCH, XCH)], bufs[c % 2], lsem)
    for k in range(K):
        h_scat[NXC - 2][k].wait()
        h_scat[NXC - 1][k].wait()

    for k in range(K):
        for c in range(NXC):
            widx = (t0 + c * XCH + iota) * 4 + k
            wk_v[...] = plsc.load_gather(wv_v, [widx])
            pltpu.sync_copy(wk_v, ws_out.at[pos2_v.at[k * NXC + c]])


def _gemm_body(be_ref, nb_ref, ws_ref, xs_ref, gup_ref, down_ref, y_ref):
    b = pl.program_id(0)

    @pl.when(b < nb_ref[0])
    def _():
        x = xs_ref[...].astype(jnp.bfloat16)
        gu = jax.lax.dot_general(
            x, gup_ref[0], (((1,), (1,)), ((), ())),
            preferred_element_type=jnp.float32)
        gate = gu[:, :DFF]
        up = gu[:, DFF:]
        h = (gate * jax.lax.logistic(gate) * up).astype(jnp.bfloat16)
        dout = jax.lax.dot_general(
            h, down_ref[0], (((1,), (1,)), ((), ())),
            preferred_element_type=jnp.float32)
        y_ref[...] = ws_ref[...] * dout


TG = 4
NG = TPW // TG


def _combine_body(y_hbm, inv_hbm, out_hbm,
                  inv_v, yrows_a, yrows_b, outbuf_v, sem):
    cid = lax.axis_index("c")
    sid = lax.axis_index("s")
    wid = sid * NC + cid
    t0 = wid * TPW
    ybufs = [yrows_a, yrows_b]

    pltpu.sync_copy(inv_hbm.at[pl.ds(wid * K * TPW, K * TPW)], inv_v)
    h = [None] * NG
    for g in range(2):
        h[g] = pltpu.async_copy(
            y_hbm.at[inv_v.at[pl.ds(g * TG * K, TG * K)]], ybufs[g], sem)
    for g in range(NG):
        h[g].wait()
        ybuf = ybufs[g % 2]

        def acc_step(m, carry):
            ii = m >> 7
            q = m & 127
            a = (ybuf[ii * 4 + 0, pl.ds(q * 16, 16)]
                 + ybuf[ii * 4 + 1, pl.ds(q * 16, 16)]
                 + ybuf[ii * 4 + 2, pl.ds(q * 16, 16)]
                 + ybuf[ii * 4 + 3, pl.ds(q * 16, 16)])
            outbuf_v[ii, pl.ds(q * 16, 16)] = a
            return carry
        lax.fori_loop(0, TG * (D // 16), acc_step, 0)
        if g + 2 < NG:
            h[g + 2] = pltpu.async_copy(
                y_hbm.at[inv_v.at[pl.ds((g + 2) * TG * K, TG * K)]],
                ybufs[g % 2], sem)
        pltpu.sync_copy(outbuf_v, out_hbm.at[pl.ds(t0 + g * TG, TG), :])


def _sc_mesh():
    return plsc.VectorSubcoreMesh(core_axis_name="c", subcore_axis_name="s",
                                  num_cores=NC, num_subcores=NS)


def _make_dispatch():
    mesh = _sc_mesh()
    return functools.partial(
        pl.kernel,
        out_type=(jax.ShapeDtypeStruct((P, D), jnp.float32),
                  jax.ShapeDtypeStruct((P,), jnp.float32),
                  jax.ShapeDtypeStruct((NW * K * TPW,), jnp.int32),
                  jax.ShapeDtypeStruct((MAXB,), jnp.int32),
                  jax.ShapeDtypeStruct((16,), jnp.int32)),
        mesh=mesh,
        compiler_params=pltpu.CompilerParams(needs_layout_passes=False),
        scratch_types=[
            pltpu.VMEM((S * K,), jnp.int32),
            pltpu.VMEM((S * K,), jnp.float32),
            pltpu.VMEM((16,), jnp.int32),
            pltpu.VMEM((16,), jnp.int32),
            pltpu.VMEM((16,), jnp.int32),
            pltpu.VMEM((16,), jnp.int32),
            pltpu.VMEM((K * NXC, XCH), jnp.int32),
            pltpu.VMEM((K * TPW,), jnp.int32),
            pltpu.VMEM((16,), jnp.float32),
            pltpu.VMEM((XCH, D), jnp.float32),
            pltpu.VMEM((XCH, D), jnp.float32),
            pltpu.VMEM((MAXB,), jnp.int32),
            pltpu.VMEM((16,), jnp.int32),
            pltpu.SemaphoreType.DMA,
            pltpu.SemaphoreType.DMA,
        ],
    )(_dispatch_body)


def _make_combine():
    mesh = _sc_mesh()
    return functools.partial(
        pl.kernel,
        out_type=jax.ShapeDtypeStruct((S, D), jnp.float32),
        mesh=mesh,
        compiler_params=pltpu.CompilerParams(needs_layout_passes=False),
        scratch_types=[
            pltpu.VMEM((K * TPW,), jnp.int32),
            pltpu.VMEM((TG * K, D), jnp.float32),
            pltpu.VMEM((TG * K, D), jnp.float32),
            pltpu.VMEM((TG, D), jnp.float32),
            pltpu.SemaphoreType.DMA,
        ],
    )(_combine_body)


def kernel(hidden_states, gate_weight, gate_up_proj, down_proj):
    b, s, d = hidden_states.shape
    x = hidden_states.reshape(s, d)

    eids, wv = pl.pallas_call(
        _router_body,
        out_shape=(jax.ShapeDtypeStruct((S, K), jnp.int32),
                   jax.ShapeDtypeStruct((S, K), jnp.float32)),
    )(x, gate_weight)

    eids_f = eids.reshape(S * K)
    wv_f = wv.reshape(S * K)

    xs, ws, inv, be, nb = _make_dispatch()(eids_f, wv_f, x)

    gupb = gate_up_proj.astype(jnp.bfloat16)
    downb = down_proj.astype(jnp.bfloat16)
    ws2 = ws.reshape(P, 1)

    grid_spec = pltpu.PrefetchScalarGridSpec(
        num_scalar_prefetch=2,
        grid=(MAXB,),
        in_specs=[
            pl.BlockSpec((BLK, 1),
                         lambda i, be, nb: (jnp.minimum(i, nb[0] - 1), 0)),
            pl.BlockSpec((BLK, D),
                         lambda i, be, nb: (jnp.minimum(i, nb[0] - 1), 0)),
            pl.BlockSpec((1, 2 * DFF, D), lambda i, be, nb: (be[i], 0, 0)),
            pl.BlockSpec((1, D, DFF), lambda i, be, nb: (be[i], 0, 0)),
        ],
        out_specs=pl.BlockSpec(
            (BLK, D), lambda i, be, nb: (jnp.minimum(i, nb[0] - 1), 0)),
    )
    y = pl.pallas_call(
        _gemm_body,
        grid_spec=grid_spec,
        out_shape=jax.ShapeDtypeStruct((P, D), jnp.float32),
    )(be, nb, ws2, xs, gupb, downb)

    out = _make_combine()(y, inv)
    return out.reshape(b, s, d)


# speedup vs baseline: 2.6339x; 1.0108x over previous
import functools

import jax
import jax.numpy as jnp
from jax import lax
from jax.experimental import pallas as pl
from jax.experimental.pallas import tpu as pltpu
from jax.experimental.pallas import tpu_sc as plsc

E = 16
K = 4
S = 2048
D = 2048
DFF = 768

NC = 2
NS = 16
NW = NC * NS
TPW = S // NW
BLK = 256
P = 12288
MAXB = P // BLK
DCH = 256
NDC = D // DCH


def _router_body(x_ref, gw_ref, eid_ref, wv_ref):
    x = x_ref[...]
    logits = jax.lax.dot_general(
        x, gw_ref[...], (((1,), (1,)), ((), ())),
        preferred_element_type=jnp.float32)
    m = jnp.max(logits, axis=-1, keepdims=True)
    ex = jnp.exp(logits - m)
    probs = ex / jnp.sum(ex, axis=-1, keepdims=True)
    p = probs
    idx = jax.lax.broadcasted_iota(jnp.int32, probs.shape, 1)
    ids = []
    vals = []
    for _ in range(K):
        cur = jnp.max(p, axis=-1, keepdims=True)
        amax = jnp.min(jnp.where(p == cur, idx, E), axis=-1, keepdims=True)
        mask = idx == amax
        p = jnp.where(mask, -jnp.inf, p)
        ids.append(amax)
        vals.append(cur)
    ids = jnp.concatenate(ids, axis=1)
    vals = jnp.concatenate(vals, axis=1)
    denom = jnp.clip(jnp.sum(vals, axis=-1, keepdims=True), 1e-12, None)
    eid_ref[...] = ids
    wv_ref[...] = vals / denom


XCH = 16
NXC = TPW // XCH


def _hist_body(eids_hbm, histw_out, eids_v, hist_v):
    cid = lax.axis_index("c")
    sid = lax.axis_index("s")
    wid = sid * NC + cid
    iota = lax.iota(jnp.int32, 16)
    m4 = iota < 4
    lane4 = iota & 3
    ones = jnp.ones((16,), jnp.int32)

    pltpu.sync_copy(eids_hbm.at[pl.ds(wid * K * TPW, K * TPW)], eids_v)
    hist_v[...] = jnp.zeros((16,), jnp.int32)

    def hist_step(i, carry):
        idvec = plsc.load_gather(eids_v, [i * 4 + lane4], mask=m4) & 15
        plsc.addupdate_scatter(hist_v, [idvec], ones, mask=m4)
        return carry
    lax.fori_loop(0, TPW, hist_step, 0)
    pltpu.sync_copy(hist_v, histw_out.at[wid])


def _dispatch_body(eids_hbm, wv_hbm, xb_hbm, histw_hbm,
                   xs_out, ws_out, inv_out, be_out, nb_out,
                   eids_v, wv_v, histw_v, wbase_v, ps_v,
                   pos2_v, posflat_v, wk_v, rows_a, rows_b, blk_v, nb_v,
                   lsem, ssem):
    cid = lax.axis_index("c")
    sid = lax.axis_index("s")
    wid = sid * NC + cid
    t0 = wid * TPW
    iota = lax.iota(jnp.int32, 16)
    m4 = iota < 4
    lane4 = iota & 3
    ones = jnp.ones((16,), jnp.int32)
    bufs = [rows_a, rows_b]

    h_load = [None] * NXC
    for c in range(2):
        h_load[c] = pltpu.async_copy(
            xb_hbm.at[pl.ds(t0 + c * XCH, XCH)], bufs[c], lsem)

    pltpu.sync_copy(eids_hbm.at[pl.ds(wid * K * TPW, K * TPW)], eids_v)
    pltpu.sync_copy(wv_hbm.at[pl.ds(wid * K * TPW, K * TPW)], wv_v)
    pltpu.sync_copy(histw_hbm, histw_v)

    total = jnp.zeros((16,), jnp.int32)
    pre = jnp.zeros((16,), jnp.int32)
    for w2 in range(NW):
        row = histw_v[w2]
        total = total + row
        pre = pre + jnp.where((iota * 0 + w2) < wid, row, 0)

    padded = ((total + (BLK - 1)) >> 8) << 8
    incl = plsc.cumsum(padded)
    ps = incl - padded
    ps_v[...] = ps
    wbase_v[...] = ps + pre

    def pos_step(ii, carry):
        idvec = plsc.load_gather(eids_v, [ii * 4 + lane4], mask=m4) & 15
        pos4 = plsc.load_gather(wbase_v, [idvec], mask=m4)
        plsc.store_scatter(pos2_v,
                           [lane4 * NXC + ii // XCH, iota * 0 + (ii & (XCH - 1))],
                           pos4, mask=m4)
        plsc.store_scatter(posflat_v, [ii * 4 + lane4], pos4, mask=m4)
        plsc.addupdate_scatter(wbase_v, [idvec], ones, mask=m4)
        return carry
    lax.fori_loop(0, TPW, pos_step, 0)

    pltpu.sync_copy(posflat_v, inv_out.at[pl.ds(wid * K * TPW, K * TPW)])

    @pl.when(wid == 0)
    def _():
        for jj in range(MAXB // 16):
            bvals = (jj * 16 + iota) * BLK
            cnt = jnp.zeros((16,), jnp.int32)
            for e in range(E):
                se = jnp.sum(jnp.where(iota == e, ps, 0))
                cnt += (bvals >= se).astype(jnp.int32)
            blk_v[pl.ds(jj * 16, 16)] = cnt - 1
        pltpu.sync_copy(blk_v, be_out)
        nb_scalar = jnp.max(incl) // BLK
        nb_v[...] = iota * 0 + nb_scalar
        pltpu.sync_copy(nb_v, nb_out)

    h_scat = [[None] * K for _ in range(NXC)]
    for c in range(NXC):
        h_load[c].wait()
        for k in range(K):
            h_scat[c][k] = pltpu.async_copy(
                bufs[c % 2], xs_out.at[pos2_v.at[k * NXC + c]], ssem)
        if c + 2 < NXC:
            for k in range(K):
                h_scat[c][k].wait()
            h_load[c + 2] = pltpu.async_copy(
                xb_hbm.at[pl.ds(t0 + (c + 2) * XCH, XCH)], bufs[c % 2], lsem)
    for k in range(K):
        h_scat[NXC - 2][k].wait()
        h_scat[NXC - 1][k].wait()

    for k in range(K):
        for c in range(NXC):
            widx = (c * XCH + iota) * 4 + k
            wk_v[...] = plsc.load_gather(wv_v, [widx])
            pltpu.sync_copy(wk_v, ws_out.at[pos2_v.at[k * NXC + c]])


def _gemm_body(be_ref, nb_ref, ws_ref, xs_ref, gup_ref, down_ref, y_ref):
    b = pl.program_id(0)

    @pl.when(b < nb_ref[0])
    def _():
        x = xs_ref[...].astype(jnp.bfloat16)
        gu = jax.lax.dot_general(
            x, gup_ref[0], (((1,), (1,)), ((), ())),
            preferred_element_type=jnp.float32)
        gate = gu[:, :DFF]
        up = gu[:, DFF:]
        h = (gate * jax.lax.logistic(gate) * up).astype(jnp.bfloat16)
        dout = jax.lax.dot_general(
            h, down_ref[0], (((1,), (1,)), ((), ())),
            preferred_element_type=jnp.float32)
        y_ref[...] = ws_ref[...] * dout


TG = 4
NG = TPW // TG


def _combine_body(y_hbm, inv_hbm, out_hbm,
                  inv_v, yrows_a, yrows_b, outbuf_v, sem):
    cid = lax.axis_index("c")
    sid = lax.axis_index("s")
    wid = sid * NC + cid
    t0 = wid * TPW
    ybufs = [yrows_a, yrows_b]

    pltpu.sync_copy(inv_hbm.at[pl.ds(wid * K * TPW, K * TPW)], inv_v)
    h = [None] * NG
    for g in range(2):
        h[g] = pltpu.async_copy(
            y_hbm.at[inv_v.at[pl.ds(g * TG * K, TG * K)]], ybufs[g], sem)
    for g in range(NG):
        h[g].wait()
        ybuf = ybufs[g % 2]

        def acc_step(m, carry):
            ii = m >> 7
            q = m & 127
            a = (ybuf[ii * 4 + 0, pl.ds(q * 16, 16)]
                 + ybuf[ii * 4 + 1, pl.ds(q * 16, 16)]
                 + ybuf[ii * 4 + 2, pl.ds(q * 16, 16)]
                 + ybuf[ii * 4 + 3, pl.ds(q * 16, 16)])
            outbuf_v[ii, pl.ds(q * 16, 16)] = a
            return carry
        lax.fori_loop(0, TG * (D // 16), acc_step, 0)
        if g + 2 < NG:
            h[g + 2] = pltpu.async_copy(
                y_hbm.at[inv_v.at[pl.ds((g + 2) * TG * K, TG * K)]],
                ybufs[g % 2], sem)
        pltpu.sync_copy(outbuf_v, out_hbm.at[pl.ds(t0 + g * TG, TG), :])


def _sc_mesh():
    return plsc.VectorSubcoreMesh(core_axis_name="c", subcore_axis_name="s",
                                  num_cores=NC, num_subcores=NS)


def _make_dispatch():
    mesh = _sc_mesh()
    return functools.partial(
        pl.kernel,
        out_type=(jax.ShapeDtypeStruct((P, D), jnp.float32),
                  jax.ShapeDtypeStruct((P,), jnp.float32),
                  jax.ShapeDtypeStruct((NW * K * TPW,), jnp.int32),
                  jax.ShapeDtypeStruct((MAXB,), jnp.int32),
                  jax.ShapeDtypeStruct((16,), jnp.int32)),
        mesh=mesh,
        compiler_params=pltpu.CompilerParams(needs_layout_passes=False),
        scratch_types=[
            pltpu.VMEM((K * TPW,), jnp.int32),
            pltpu.VMEM((K * TPW,), jnp.float32),
            pltpu.VMEM((NW, 16), jnp.int32),
            pltpu.VMEM((16,), jnp.int32),
            pltpu.VMEM((16,), jnp.int32),
            pltpu.VMEM((K * NXC, XCH), jnp.int32),
            pltpu.VMEM((K * TPW,), jnp.int32),
            pltpu.VMEM((16,), jnp.float32),
            pltpu.VMEM((XCH, D), jnp.float32),
            pltpu.VMEM((XCH, D), jnp.float32),
            pltpu.VMEM((MAXB,), jnp.int32),
            pltpu.VMEM((16,), jnp.int32),
            pltpu.SemaphoreType.DMA,
            pltpu.SemaphoreType.DMA,
        ],
    )(_dispatch_body)


def _make_combine():
    mesh = _sc_mesh()
    return functools.partial(
        pl.kernel,
        out_type=jax.ShapeDtypeStruct((S, D), jnp.float32),
        mesh=mesh,
        compiler_params=pltpu.CompilerParams(needs_layout_passes=False),
        scratch_types=[
            pltpu.VMEM((K * TPW,), jnp.int32),
            pltpu.VMEM((TG * K, D), jnp.float32),
            pltpu.VMEM((TG * K, D), jnp.float32),
            pltpu.VMEM((TG, D), jnp.float32),
            pltpu.SemaphoreType.DMA,
        ],
    )(_combine_body)


def kernel(hidden_states, gate_weight, gate_up_proj, down_proj):
    b, s, d = hidden_states.shape
    x = hidden_states.reshape(s, d)

    eids, wv = pl.pallas_call(
        _router_body,
        out_shape=(jax.ShapeDtypeStruct((S, K), jnp.int32),
                   jax.ShapeDtypeStruct((S, K), jnp.float32)),
    )(x, gate_weight)

    eids_f = eids.reshape(S * K)
    wv_f = wv.reshape(S * K)

    mesh = _sc_mesh()
    histw = functools.partial(
        pl.kernel,
        out_type=jax.ShapeDtypeStruct((NW, 16), jnp.int32),
        mesh=mesh,
        compiler_params=pltpu.CompilerParams(needs_layout_passes=False),
        scratch_types=[
            pltpu.VMEM((K * TPW,), jnp.int32),
            pltpu.VMEM((16,), jnp.int32),
        ],
    )(_hist_body)(eids_f)

    xs, ws, inv, be, nb = _make_dispatch()(eids_f, wv_f, x, histw)

    gupb = gate_up_proj.astype(jnp.bfloat16)
    downb = down_proj.astype(jnp.bfloat16)
    ws2 = ws.reshape(P, 1)

    grid_spec = pltpu.PrefetchScalarGridSpec(
        num_scalar_prefetch=2,
        grid=(MAXB,),
        in_specs=[
            pl.BlockSpec((BLK, 1),
                         lambda i, be, nb: (jnp.minimum(i, nb[0] - 1), 0)),
            pl.BlockSpec((BLK, D),
                         lambda i, be, nb: (jnp.minimum(i, nb[0] - 1), 0)),
            pl.BlockSpec((1, 2 * DFF, D), lambda i, be, nb: (be[i], 0, 0)),
            pl.BlockSpec((1, D, DFF), lambda i, be, nb: (be[i], 0, 0)),
        ],
        out_specs=pl.BlockSpec(
            (BLK, D), lambda i, be, nb: (jnp.minimum(i, nb[0] - 1), 0)),
    )
    y = pl.pallas_call(
        _gemm_body,
        grid_spec=grid_spec,
        out_shape=jax.ShapeDtypeStruct((P, D), jnp.float32),
    )(be, nb, ws2, xs, gupb, downb)

    out = _make_combine()(y, inv)
    return out.reshape(b, s, d)
